# Initial kernel scaffold; baseline (speedup 1.0000x reference)
import jax
import jax.numpy as jnp
from jax.experimental import pallas as pl


def kernel(x_author, x_paper, params, edge_index_writes, edge_index_rev, edge_label_index_writes, edge_label_index_rev):
    raise NotImplementedError("write your pallas kernel here")


# speedup vs baseline: 32.1208x; 32.1208x over previous
import functools

import jax
import jax.numpy as jnp
from jax import lax
from jax.experimental import pallas as pl
from jax.experimental.pallas import tpu as pltpu
from jax.experimental.pallas import tpu_sc as plsc

_EPS = 1e-16
_CH = 128
_NSUB = 16
_NCORE = 2


def _proj_body(norm, xa_ref, da_ref, xp_ref, dp_ref, wa_ref, ba_ref, wp_ref,
               bp_ref, aa_ref, ap_ref, ha_ref, hp_ref, ala_ref, alp_ref):
    xa = xa_ref[...]
    xp = xp_ref[...]
    if norm:
        xa = jnp.maximum(xa / (da_ref[...] + _EPS), 0.0)
        xp = jnp.maximum(xp / (dp_ref[...] + _EPS), 0.0)
    ha = jnp.dot(xa, wa_ref[...], preferred_element_type=jnp.float32) + ba_ref[...]
    hp = jnp.dot(xp, wp_ref[...], preferred_element_type=jnp.float32) + bp_ref[...]
    ha_ref[...] = ha
    hp_ref[...] = hp
    ala_ref[...] = jnp.dot(ha, aa_ref[...], preferred_element_type=jnp.float32)
    alp_ref[...] = jnp.dot(hp, ap_ref[...], preferred_element_type=jnp.float32)


def _tc_proj(norm, xa, da, xp, dp, wa, ba, wp, bp, aa, ap):
    n, c = xa.shape
    f = wa.shape[1]
    bn = 2000
    grid = (n // bn,)
    xmap = lambda i: (i, 0)
    wmap = lambda i: (0, 0)
    in_specs = [
        pl.BlockSpec((bn, c), xmap),
        pl.BlockSpec((bn, 1), xmap),
        pl.BlockSpec((bn, c), xmap),
        pl.BlockSpec((bn, 1), xmap),
        pl.BlockSpec((c, f), wmap),
        pl.BlockSpec((1, f), wmap),
        pl.BlockSpec((c, f), wmap),
        pl.BlockSpec((1, f), wmap),
        pl.BlockSpec((f, 2), wmap),
        pl.BlockSpec((f, 2), wmap),
    ]
    out_specs = [
        pl.BlockSpec((bn, f), xmap),
        pl.BlockSpec((bn, f), xmap),
        pl.BlockSpec((bn, 2), xmap),
        pl.BlockSpec((bn, 2), xmap),
    ]
    out_shape = [
        jax.ShapeDtypeStruct((n, f), jnp.float32),
        jax.ShapeDtypeStruct((n, f), jnp.float32),
        jax.ShapeDtypeStruct((n, 2), jnp.float32),
        jax.ShapeDtypeStruct((n, 2), jnp.float32),
    ]
    return pl.pallas_call(
        functools.partial(_proj_body, norm),
        grid=grid, in_specs=in_specs, out_specs=out_specs, out_shape=out_shape,
    )(xa, da, xp, dp, wa, ba, wp, bp, aa, ap)


def _final_body(xa_ref, da_ref, xp_ref, dp_ref, w_ref, b_ref, mw_ref, mr_ref,
                oa_ref, op_ref, up_ref, ua_ref):
    xa = jnp.maximum(xa_ref[...] / (da_ref[...] + _EPS), 0.0)
    xp = jnp.maximum(xp_ref[...] / (dp_ref[...] + _EPS), 0.0)
    oa = jnp.dot(xa, w_ref[...], preferred_element_type=jnp.float32) + b_ref[...]
    op = jnp.dot(xp, w_ref[...], preferred_element_type=jnp.float32) + b_ref[...]
    oa_ref[...] = oa
    op_ref[...] = op
    up_ref[...] = jnp.dot(op, mw_ref[...], preferred_element_type=jnp.float32)
    ua_ref[...] = jnp.dot(oa, mr_ref[...], preferred_element_type=jnp.float32)


def _tc_final(xa, da, xp, dp, w, b, mw, mr):
    n, c = xa.shape
    bn = 2000
    grid = (n // bn,)
    xmap = lambda i: (i, 0)
    wmap = lambda i: (0, 0)
    in_specs = [
        pl.BlockSpec((bn, c), xmap),
        pl.BlockSpec((bn, 1), xmap),
        pl.BlockSpec((bn, c), xmap),
        pl.BlockSpec((bn, 1), xmap),
        pl.BlockSpec((c, 2), wmap),
        pl.BlockSpec((1, 2), wmap),
        pl.BlockSpec((2, 2), wmap),
        pl.BlockSpec((2, 2), wmap),
    ]
    out_specs = [pl.BlockSpec((bn, 2), xmap)] * 4
    out_shape = [jax.ShapeDtypeStruct((n, 2), jnp.float32)] * 4
    return pl.pallas_call(
        _final_body,
        grid=grid, in_specs=in_specs, out_specs=out_specs, out_shape=out_shape,
    )(xa, da, xp, dp, w, b, mw, mr)


@functools.lru_cache(maxsize=None)
def _mp_kernel(F, EP, NP):
    ET = EP // _NSUB
    nchunk = ET // _CH
    rows_per_tile = NP // _NSUB
    nout = rows_per_tile // _CH
    mesh = plsc.VectorSubcoreMesh(core_axis_name="c", subcore_axis_name="s",
                                  num_cores=_NCORE, num_subcores=_NSUB)

    def body(ha, hp, roww, colw, rowr, colr, asw, adw, asr, adr,
             aggp, denp, agga, dena,
             asrc_v, adst_v, row_v, col_v, w_v, rows_v, out_sp, den_sp, sem):
        s = lax.axis_index("s")
        c = lax.axis_index("c")
        zero16 = jnp.zeros((16,), jnp.float32)

        def run(h_hbm, row_hbm, col_hbm, as_hbm, ad_hbm, agg_hbm, den_hbm):
            pltpu.sync_copy(as_hbm, asrc_v)
            pltpu.sync_copy(ad_hbm, adst_v)

            def zrow(e, _):
                for f in range(F // 16):
                    rows_v[e, f * 16:(f + 1) * 16] = zero16
                return 0
            lax.fori_loop(0, _CH, zrow, 0)
            for g in range(_CH // 16):
                w_v[g * 16:(g + 1) * 16] = zero16
            for k in range(nout):
                off = s * rows_per_tile + k * _CH
                pltpu.sync_copy(rows_v, out_sp.at[pl.ds(off, _CH)])
                pltpu.sync_copy(w_v, den_sp.at[pl.ds(off, _CH)])
            plsc.subcore_barrier()

            def chunk(g, _):
                base = s * ET + g * _CH
                pltpu.sync_copy(row_hbm.at[pl.ds(base, _CH)], row_v)
                pltpu.sync_copy(col_hbm.at[pl.ds(base, _CH)], col_v)
                gat = pltpu.async_copy(h_hbm.at[row_v], rows_v, sem)
                for gg in range(_CH // 16):
                    r16 = row_v[gg * 16:(gg + 1) * 16]
                    c16 = col_v[gg * 16:(gg + 1) * 16]
                    a = (plsc.load_gather(asrc_v, [r16])
                         + plsc.load_gather(adst_v, [c16]))
                    a = jnp.where(a > 0.0, a, 0.2 * a)
                    w_v[gg * 16:(gg + 1) * 16] = jnp.exp(a)
                pltpu.sync_copy(w_v, den_sp.at[col_v], add=True)
                gat.wait()

                def scale(g2, _):
                    w16 = w_v[pl.ds(g2 * 16, 16)]
                    for e in range(16):
                        sw = w16[e]
                        idx = g2 * 16 + e
                        for f in range(F // 16):
                            rows_v[idx, f * 16:(f + 1) * 16] = (
                                rows_v[idx, f * 16:(f + 1) * 16] * sw)
                    return 0
                lax.fori_loop(0, _CH // 16, scale, 0)
                pltpu.sync_copy(rows_v, out_sp.at[col_v], add=True)
                return 0
            lax.fori_loop(0, nchunk, chunk, 0)
            plsc.subcore_barrier()

            for k in range(nout):
                off = s * rows_per_tile + k * _CH
                pltpu.sync_copy(out_sp.at[pl.ds(off, _CH)], rows_v)
                pltpu.sync_copy(rows_v, agg_hbm.at[pl.ds(off, _CH)])
                pltpu.sync_copy(den_sp.at[pl.ds(off, _CH)], w_v)
                pltpu.sync_copy(w_v, den_hbm.at[pl.ds(off, _CH)])

        @pl.when(c == 0)
        def _():
            run(ha, roww, colw, asw, adw, aggp, denp)

        @pl.when(c == 1)
        def _():
            run(hp, rowr, colr, asr, adr, agga, dena)

    return pl.kernel(
        body,
        out_type=[
            jax.ShapeDtypeStruct((NP, F), jnp.float32),
            jax.ShapeDtypeStruct((NP,), jnp.float32),
            jax.ShapeDtypeStruct((NP, F), jnp.float32),
            jax.ShapeDtypeStruct((NP,), jnp.float32),
        ],
        mesh=mesh,
        compiler_params=pltpu.CompilerParams(needs_layout_passes=False, use_tc_tiling_on_sc=False),
        scratch_types=[
            pltpu.VMEM((NP,), jnp.float32),
            pltpu.VMEM((NP,), jnp.float32),
            pltpu.VMEM((_CH,), jnp.int32),
            pltpu.VMEM((_CH,), jnp.int32),
            pltpu.VMEM((_CH,), jnp.float32),
            pltpu.VMEM((_CH, F), jnp.float32),
            pltpu.VMEM_SHARED((NP, F), jnp.float32),
            pltpu.VMEM_SHARED((NP,), jnp.float32),
            pltpu.SemaphoreType.DMA,
        ],
    )


@functools.lru_cache(maxsize=None)
def _score_kernel(N, LP):
    per = LP // _NSUB
    mesh = plsc.VectorSubcoreMesh(core_axis_name="c", subcore_axis_name="s",
                                  num_cores=_NCORE, num_subcores=_NSUB)

    def body(oa, up, op, ua, eliw, elir, out,
             head_v, tail_v, src_v, dst_v, res_v):
        s = lax.axis_index("s")
        c = lax.axis_index("c")

        def run(h_hbm, t_hbm, eli_hbm, out_row):
            pltpu.sync_copy(h_hbm, head_v)
            pltpu.sync_copy(t_hbm, tail_v)
            base = s * per
            pltpu.sync_copy(eli_hbm.at[0, pl.ds(base, per)], src_v)
            pltpu.sync_copy(eli_hbm.at[1, pl.ds(base, per)], dst_v)
            def grp(g, _):
                s16 = src_v[pl.ds(g * 16, 16)] * 2
                d16 = dst_v[pl.ds(g * 16, 16)] * 2
                h0 = plsc.load_gather(head_v, [s16])
                h1 = plsc.load_gather(head_v, [s16 + 1])
                u0 = plsc.load_gather(tail_v, [d16])
                u1 = plsc.load_gather(tail_v, [d16 + 1])
                res_v[pl.ds(g * 16, 16)] = h0 * u0 + h1 * u1
                return 0
            lax.fori_loop(0, per // 16, grp, 0)
            pltpu.sync_copy(res_v, out.at[out_row, pl.ds(base, per)])

        @pl.when(c == 0)
        def _():
            run(oa, up, eliw, 0)

        @pl.when(c == 1)
        def _():
            run(op, ua, elir, 1)

    return pl.kernel(
        body,
        out_type=jax.ShapeDtypeStruct((2, LP), jnp.float32),
        mesh=mesh,
        compiler_params=pltpu.CompilerParams(needs_layout_passes=False, use_tc_tiling_on_sc=False),
        scratch_types=[
            pltpu.VMEM((2 * N,), jnp.float32),
            pltpu.VMEM((2 * N,), jnp.float32),
            pltpu.VMEM((per,), jnp.int32),
            pltpu.VMEM((per,), jnp.int32),
            pltpu.VMEM((per,), jnp.float32),
        ],
    )


def _round_up(x, m):
    return (x + m - 1) // m * m


def _pad_rows(x, np_):
    return jnp.concatenate(
        [x, jnp.zeros((np_ - x.shape[0], x.shape[1]), x.dtype)])


def _pad_alpha(v, np_):
    return jnp.concatenate(
        [v, jnp.full((np_ - v.shape[0],), -1e30, jnp.float32)])


def _pad_edges(ei, ep, n, np_):
    e = ei.shape[1]
    fill = n + (jnp.arange(ep - e, dtype=jnp.int32) % (np_ - n))
    return (jnp.concatenate([ei[0], fill]), jnp.concatenate([ei[1], fill]))


def _layer_args(lp):
    wa = lp["proj"]["author"]["W"].T
    wp = lp["proj"]["paper"]["W"].T
    ba = lp["proj"]["author"]["b"][None, :]
    bp = lp["proj"]["paper"]["b"][None, :]
    aa = jnp.stack([lp["att"]["writes"]["src"], lp["att"]["rev"]["dst"]], axis=1)
    ap = jnp.stack([lp["att"]["writes"]["dst"], lp["att"]["rev"]["src"]], axis=1)
    return wa, ba, wp, bp, aa, ap


def kernel(x_author, x_paper, params, edge_index_writes, edge_index_rev,
           edge_label_index_writes, edge_label_index_rev):
    n = x_author.shape[0]
    e = edge_index_writes.shape[1]
    l = edge_label_index_writes.shape[1]
    np_ = _round_up(n + 1, _NSUB * _CH)
    ep = _round_up(e, _NSUB * _CH)
    lp_ = _round_up(l, _NSUB * 16)

    ones = jnp.ones((n, 1), jnp.float32)

    def mp_layer(h_a, h_p, al_a, al_p, F):
        ha_p = _pad_rows(h_a, np_)
        hp_p = _pad_rows(h_p, np_)
        asw = _pad_alpha(al_a[:, 0], np_)
        adw = _pad_alpha(al_p[:, 0], np_)
        asr = _pad_alpha(al_p[:, 1], np_)
        adr = _pad_alpha(al_a[:, 1], np_)
        roww, colw = _pad_edges(edge_index_writes, ep, n, np_)
        rowr, colr = _pad_edges(edge_index_rev, ep, n, np_)
        aggp, denp, agga, dena = _mp_kernel(F, ep, np_)(
            ha_p, hp_p, roww, colw, rowr, colr, asw, adw, asr, adr)
        return (agga[:n], dena[:n, None], aggp[:n], denp[:n, None])

    h_a, h_p, al_a, al_p = _tc_proj(False, x_author, ones, x_paper, ones,
                                    *_layer_args(params["l1"]))
    agga, dena, aggp, denp = mp_layer(h_a, h_p, al_a, al_p, h_a.shape[1])

    h_a, h_p, al_a, al_p = _tc_proj(True, agga, dena, aggp, denp,
                                    *_layer_args(params["l2"]))
    agga, dena, aggp, denp = mp_layer(h_a, h_p, al_a, al_p, h_a.shape[1])

    r = params["rel_emb"]
    mw = jnp.stack([jnp.stack([r[0, 0], -r[0, 1]]),
                    jnp.stack([r[0, 1], r[0, 0]])])
    mr = jnp.stack([jnp.stack([r[1, 0], -r[1, 1]]),
                    jnp.stack([r[1, 1], r[1, 0]])])
    oa, op, up, ua = _tc_final(agga, dena, aggp, denp,
                               params["post_W"].T, params["post_b"][None, :],
                               mw, mr)

    pad = jnp.zeros((2, lp_ - l), jnp.int32)
    eliw = jnp.concatenate([edge_label_index_writes, pad], axis=1)
    elir = jnp.concatenate([edge_label_index_rev, pad], axis=1)
    scores = _score_kernel(n, lp_)(oa.reshape(-1), up.reshape(-1),
                                   op.reshape(-1), ua.reshape(-1), eliw, elir)
    return scores[:, :l]


# speedup vs baseline: 51.4006x; 1.6002x over previous
import functools

import jax
import jax.numpy as jnp
from jax import lax
from jax.experimental import pallas as pl
from jax.experimental.pallas import tpu as pltpu
from jax.experimental.pallas import tpu_sc as plsc

_EPS = 1e-16
_CH = 128
_NSUB = 16
_NCORE = 2


def _proj_body(norm, xa_ref, da_ref, xp_ref, dp_ref, wa_ref, ba_ref, wp_ref,
               bp_ref, aa_ref, ap_ref, ha_ref, hp_ref, ala_ref, alp_ref):
    xa = xa_ref[...]
    xp = xp_ref[...]
    if norm:
        xa = jnp.maximum(xa / (da_ref[...] + _EPS), 0.0)
        xp = jnp.maximum(xp / (dp_ref[...] + _EPS), 0.0)
    ha = jnp.dot(xa, wa_ref[...], preferred_element_type=jnp.float32) + ba_ref[...]
    hp = jnp.dot(xp, wp_ref[...], preferred_element_type=jnp.float32) + bp_ref[...]
    ha_ref[...] = ha
    hp_ref[...] = hp
    ala_ref[...] = jnp.dot(ha, aa_ref[...], preferred_element_type=jnp.float32)
    alp_ref[...] = jnp.dot(hp, ap_ref[...], preferred_element_type=jnp.float32)


def _tc_proj(norm, xa, da, xp, dp, wa, ba, wp, bp, aa, ap):
    n, c = xa.shape
    f = wa.shape[1]
    bn = 2000
    grid = (n // bn,)
    xmap = lambda i: (i, 0)
    wmap = lambda i: (0, 0)
    in_specs = [
        pl.BlockSpec((bn, c), xmap),
        pl.BlockSpec((bn, 1), xmap),
        pl.BlockSpec((bn, c), xmap),
        pl.BlockSpec((bn, 1), xmap),
        pl.BlockSpec((c, f), wmap),
        pl.BlockSpec((1, f), wmap),
        pl.BlockSpec((c, f), wmap),
        pl.BlockSpec((1, f), wmap),
        pl.BlockSpec((f, 2), wmap),
        pl.BlockSpec((f, 2), wmap),
    ]
    out_specs = [
        pl.BlockSpec((bn, f), xmap),
        pl.BlockSpec((bn, f), xmap),
        pl.BlockSpec((bn, 2), xmap),
        pl.BlockSpec((bn, 2), xmap),
    ]
    out_shape = [
        jax.ShapeDtypeStruct((n, f), jnp.float32),
        jax.ShapeDtypeStruct((n, f), jnp.float32),
        jax.ShapeDtypeStruct((n, 2), jnp.float32),
        jax.ShapeDtypeStruct((n, 2), jnp.float32),
    ]
    return pl.pallas_call(
        functools.partial(_proj_body, norm),
        grid=grid, in_specs=in_specs, out_specs=out_specs, out_shape=out_shape,
    )(xa, da, xp, dp, wa, ba, wp, bp, aa, ap)


def _final_body(xa_ref, da_ref, xp_ref, dp_ref, w_ref, b_ref, mw_ref, mr_ref,
                oa_ref, op_ref, up_ref, ua_ref):
    xa = jnp.maximum(xa_ref[...] / (da_ref[...] + _EPS), 0.0)
    xp = jnp.maximum(xp_ref[...] / (dp_ref[...] + _EPS), 0.0)
    oa = jnp.dot(xa, w_ref[...], preferred_element_type=jnp.float32) + b_ref[...]
    op = jnp.dot(xp, w_ref[...], preferred_element_type=jnp.float32) + b_ref[...]
    oa_ref[...] = oa
    op_ref[...] = op
    up_ref[...] = jnp.dot(op, mw_ref[...], preferred_element_type=jnp.float32)
    ua_ref[...] = jnp.dot(oa, mr_ref[...], preferred_element_type=jnp.float32)


def _tc_final(xa, da, xp, dp, w, b, mw, mr):
    n, c = xa.shape
    bn = 2000
    grid = (n // bn,)
    xmap = lambda i: (i, 0)
    wmap = lambda i: (0, 0)
    in_specs = [
        pl.BlockSpec((bn, c), xmap),
        pl.BlockSpec((bn, 1), xmap),
        pl.BlockSpec((bn, c), xmap),
        pl.BlockSpec((bn, 1), xmap),
        pl.BlockSpec((c, 2), wmap),
        pl.BlockSpec((1, 2), wmap),
        pl.BlockSpec((2, 2), wmap),
        pl.BlockSpec((2, 2), wmap),
    ]
    out_specs = [pl.BlockSpec((bn, 2), xmap)] * 4
    out_shape = [jax.ShapeDtypeStruct((n, 2), jnp.float32)] * 4
    return pl.pallas_call(
        _final_body,
        grid=grid, in_specs=in_specs, out_specs=out_specs, out_shape=out_shape,
    )(xa, da, xp, dp, w, b, mw, mr)


@functools.lru_cache(maxsize=None)
def _mp_kernel(F, EP, NP):
    ET = EP // _NSUB
    nchunk = ET // _CH
    rows_per_tile = NP // _NSUB
    nout = rows_per_tile // _CH
    mesh = plsc.VectorSubcoreMesh(core_axis_name="c", subcore_axis_name="s",
                                  num_cores=_NCORE, num_subcores=_NSUB)

    def body(ha, hp, roww, colw, rowr, colr, asw, adw, asr, adr,
             aggp, denp, agga, dena,
             asrc_v, adst_v, row3_v, col3_v, w0_v, w1_v, rows0_v, rows1_v,
             out_sp, den_sp, sem0, sem1):
        s = lax.axis_index("s")
        c = lax.axis_index("c")
        zero16 = jnp.zeros((16,), jnp.float32)
        wv = (w0_v, w1_v)
        rv = (rows0_v, rows1_v)
        sems = (sem0, sem1)

        def run(h_hbm, row_hbm, col_hbm, as_hbm, ad_hbm, agg_hbm, den_hbm):
            pltpu.sync_copy(as_hbm, asrc_v)
            pltpu.sync_copy(ad_hbm, adst_v)
            pltpu.sync_copy(row_hbm.at[pl.ds(s * nchunk, nchunk)], row3_v)
            pltpu.sync_copy(col_hbm.at[pl.ds(s * nchunk, nchunk)], col3_v)

            def zrow(e, _):
                for f in range(F // 16):
                    rows0_v[e, f * 16:(f + 1) * 16] = zero16
                return 0
            lax.fori_loop(0, _CH, zrow, 0)
            for g in range(_CH // 16):
                w0_v[g * 16:(g + 1) * 16] = zero16
            for k in range(nout):
                off = s * rows_per_tile + k * _CH
                pltpu.sync_copy(rows0_v, out_sp.at[pl.ds(off, _CH)])
                pltpu.sync_copy(w0_v, den_sp.at[pl.ds(off, _CH)])
            plsc.subcore_barrier()

            pltpu.async_copy(h_hbm.at[row3_v.at[0, 0]], rows0_v, sem0)
            pltpu.async_copy(h_hbm.at[row3_v.at[1, 0]], rows1_v, sem1)

            def half(g, k):
                w_v = wv[k]
                rows_v = rv[k]
                sem = sems[k]
                for gg in range(_CH // 16):
                    r16 = row3_v[g, 0, gg * 16:(gg + 1) * 16]
                    c16 = col3_v[g, 0, gg * 16:(gg + 1) * 16]
                    a = (plsc.load_gather(asrc_v, [r16])
                         + plsc.load_gather(adst_v, [c16]))
                    a = jnp.where(a > 0.0, a, 0.2 * a)
                    w_v[gg * 16:(gg + 1) * 16] = jnp.exp(a)
                pltpu.sync_copy(w_v, den_sp.at[col3_v.at[g, 0]], add=True)
                pltpu.make_async_copy(h_hbm.at[row3_v.at[g, 0]], rows_v,
                                      sem).wait()

                def scale(g2, _):
                    w16 = w_v[pl.ds(g2 * 16, 16)]
                    for e in range(16):
                        sw = w16[e]
                        idx = g2 * 16 + e
                        for f in range(F // 16):
                            rows_v[idx, f * 16:(f + 1) * 16] = (
                                rows_v[idx, f * 16:(f + 1) * 16] * sw)
                    return 0
                lax.fori_loop(0, _CH // 16, scale, 0)
                pltpu.sync_copy(rows_v, out_sp.at[col3_v.at[g, 0]], add=True)

                @pl.when(g + 2 < nchunk)
                def _():
                    pltpu.async_copy(h_hbm.at[row3_v.at[g + 2, 0]], rows_v, sem)

            def chunk(i, _):
                half(2 * i, 0)
                half(2 * i + 1, 1)
                return 0
            lax.fori_loop(0, nchunk // 2, chunk, 0)
            plsc.subcore_barrier()

            for k in range(nout):
                off = s * rows_per_tile + k * _CH
                pltpu.sync_copy(out_sp.at[pl.ds(off, _CH)], rows0_v)
                pltpu.sync_copy(rows0_v, agg_hbm.at[pl.ds(off, _CH)])
                pltpu.sync_copy(den_sp.at[pl.ds(off, _CH)], w0_v)
                pltpu.sync_copy(w0_v, den_hbm.at[pl.ds(off, _CH)])

        @pl.when(c == 0)
        def _():
            run(ha, roww, colw, asw, adw, aggp, denp)

        @pl.when(c == 1)
        def _():
            run(hp, rowr, colr, asr, adr, agga, dena)

    return pl.kernel(
        body,
        out_type=[
            jax.ShapeDtypeStruct((NP, F), jnp.float32),
            jax.ShapeDtypeStruct((NP,), jnp.float32),
            jax.ShapeDtypeStruct((NP, F), jnp.float32),
            jax.ShapeDtypeStruct((NP,), jnp.float32),
        ],
        mesh=mesh,
        compiler_params=pltpu.CompilerParams(needs_layout_passes=False, use_tc_tiling_on_sc=False),
        scratch_types=[
            pltpu.VMEM((NP,), jnp.float32),
            pltpu.VMEM((NP,), jnp.float32),
            pltpu.VMEM((EP // _NSUB // _CH, 1, _CH), jnp.int32),
            pltpu.VMEM((EP // _NSUB // _CH, 1, _CH), jnp.int32),
            pltpu.VMEM((_CH,), jnp.float32),
            pltpu.VMEM((_CH,), jnp.float32),
            pltpu.VMEM((_CH, F), jnp.float32),
            pltpu.VMEM((_CH, F), jnp.float32),
            pltpu.VMEM_SHARED((NP, F), jnp.float32),
            pltpu.VMEM_SHARED((NP,), jnp.float32),
            pltpu.SemaphoreType.DMA,
            pltpu.SemaphoreType.DMA,
        ],
    )


@functools.lru_cache(maxsize=None)
def _score_kernel(N, LP):
    per = LP // _NSUB
    mesh = plsc.VectorSubcoreMesh(core_axis_name="c", subcore_axis_name="s",
                                  num_cores=_NCORE, num_subcores=_NSUB)

    def body(oa, up, op, ua, eliw, elir, out,
             head_v, tail_v, src_v, dst_v, res_v):
        s = lax.axis_index("s")
        c = lax.axis_index("c")

        def run(h_hbm, t_hbm, eli_hbm, out_row):
            pltpu.sync_copy(h_hbm, head_v)
            pltpu.sync_copy(t_hbm, tail_v)
            base = s * per
            pltpu.sync_copy(eli_hbm.at[0, pl.ds(base, per)], src_v)
            pltpu.sync_copy(eli_hbm.at[1, pl.ds(base, per)], dst_v)
            def grp(g, _):
                s16 = src_v[pl.ds(g * 16, 16)] * 2
                d16 = dst_v[pl.ds(g * 16, 16)] * 2
                h0 = plsc.load_gather(head_v, [s16])
                h1 = plsc.load_gather(head_v, [s16 + 1])
                u0 = plsc.load_gather(tail_v, [d16])
                u1 = plsc.load_gather(tail_v, [d16 + 1])
                res_v[pl.ds(g * 16, 16)] = h0 * u0 + h1 * u1
                return 0
            lax.fori_loop(0, per // 16, grp, 0)
            pltpu.sync_copy(res_v, out.at[out_row, pl.ds(base, per)])

        @pl.when(c == 0)
        def _():
            run(oa, up, eliw, 0)

        @pl.when(c == 1)
        def _():
            run(op, ua, elir, 1)

    return pl.kernel(
        body,
        out_type=jax.ShapeDtypeStruct((2, LP), jnp.float32),
        mesh=mesh,
        compiler_params=pltpu.CompilerParams(needs_layout_passes=False, use_tc_tiling_on_sc=False),
        scratch_types=[
            pltpu.VMEM((2 * N,), jnp.float32),
            pltpu.VMEM((2 * N,), jnp.float32),
            pltpu.VMEM((per,), jnp.int32),
            pltpu.VMEM((per,), jnp.int32),
            pltpu.VMEM((per,), jnp.float32),
        ],
    )


def _round_up(x, m):
    return (x + m - 1) // m * m


def _pad_rows(x, np_):
    return jnp.concatenate(
        [x, jnp.zeros((np_ - x.shape[0], x.shape[1]), x.dtype)])


def _pad_alpha(v, np_):
    return jnp.concatenate(
        [v, jnp.full((np_ - v.shape[0],), -1e30, jnp.float32)])


def _pad_edges(ei, ep, n, np_):
    e = ei.shape[1]
    fill = n + (jnp.arange(ep - e, dtype=jnp.int32) % (np_ - n))
    row = jnp.concatenate([ei[0], fill]).reshape(ep // _CH, 1, _CH)
    col = jnp.concatenate([ei[1], fill]).reshape(ep // _CH, 1, _CH)
    return row, col


def _layer_args(lp):
    wa = lp["proj"]["author"]["W"].T
    wp = lp["proj"]["paper"]["W"].T
    ba = lp["proj"]["author"]["b"][None, :]
    bp = lp["proj"]["paper"]["b"][None, :]
    aa = jnp.stack([lp["att"]["writes"]["src"], lp["att"]["rev"]["dst"]], axis=1)
    ap = jnp.stack([lp["att"]["writes"]["dst"], lp["att"]["rev"]["src"]], axis=1)
    return wa, ba, wp, bp, aa, ap


def kernel(x_author, x_paper, params, edge_index_writes, edge_index_rev,
           edge_label_index_writes, edge_label_index_rev):
    n = x_author.shape[0]
    e = edge_index_writes.shape[1]
    l = edge_label_index_writes.shape[1]
    np_ = _round_up(n + 1, _NSUB * _CH)
    ep = _round_up(e, 2 * _NSUB * _CH)
    lp_ = _round_up(l, _NSUB * 16)

    ones = jnp.ones((n, 1), jnp.float32)

    def mp_layer(h_a, h_p, al_a, al_p, F):
        ha_p = _pad_rows(h_a, np_)
        hp_p = _pad_rows(h_p, np_)
        asw = _pad_alpha(al_a[:, 0], np_)
        adw = _pad_alpha(al_p[:, 0], np_)
        asr = _pad_alpha(al_p[:, 1], np_)
        adr = _pad_alpha(al_a[:, 1], np_)
        roww, colw = _pad_edges(edge_index_writes, ep, n, np_)
        rowr, colr = _pad_edges(edge_index_rev, ep, n, np_)
        aggp, denp, agga, dena = _mp_kernel(F, ep, np_)(
            ha_p, hp_p, roww, colw, rowr, colr, asw, adw, asr, adr)
        return (agga[:n], dena[:n, None], aggp[:n], denp[:n, None])

    h_a, h_p, al_a, al_p = _tc_proj(False, x_author, ones, x_paper, ones,
                                    *_layer_args(params["l1"]))
    agga, dena, aggp, denp = mp_layer(h_a, h_p, al_a, al_p, h_a.shape[1])

    h_a, h_p, al_a, al_p = _tc_proj(True, agga, dena, aggp, denp,
                                    *_layer_args(params["l2"]))
    agga, dena, aggp, denp = mp_layer(h_a, h_p, al_a, al_p, h_a.shape[1])

    r = params["rel_emb"]
    mw = jnp.stack([jnp.stack([r[0, 0], -r[0, 1]]),
                    jnp.stack([r[0, 1], r[0, 0]])])
    mr = jnp.stack([jnp.stack([r[1, 0], -r[1, 1]]),
                    jnp.stack([r[1, 1], r[1, 0]])])
    oa, op, up, ua = _tc_final(agga, dena, aggp, denp,
                               params["post_W"].T, params["post_b"][None, :],
                               mw, mr)

    pad = jnp.zeros((2, lp_ - l), jnp.int32)
    eliw = jnp.concatenate([edge_label_index_writes, pad], axis=1)
    elir = jnp.concatenate([edge_label_index_rev, pad], axis=1)
    scores = _score_kernel(n, lp_)(oa.reshape(-1), up.reshape(-1),
                                   op.reshape(-1), ua.reshape(-1), eliw, elir)
    return scores[:, :l]


# speedup vs baseline: 58.2982x; 1.1342x over previous
import functools

import jax
import jax.numpy as jnp
from jax import lax
from jax.experimental import pallas as pl
from jax.experimental.pallas import tpu as pltpu
from jax.experimental.pallas import tpu_sc as plsc

_EPS = 1e-16
_CH = 128
_NSUB = 16
_NCORE = 2


def _proj_body(norm, xa_ref, da_ref, xp_ref, dp_ref, wa_ref, ba_ref, wp_ref,
               bp_ref, aa_ref, ap_ref, ha_ref, hp_ref, ala_ref, alp_ref):
    xa = xa_ref[...]
    xp = xp_ref[...]
    if norm:
        xa = jnp.maximum(xa / (da_ref[...] + _EPS), 0.0)
        xp = jnp.maximum(xp / (dp_ref[...] + _EPS), 0.0)
    ha = jnp.dot(xa, wa_ref[...], preferred_element_type=jnp.float32) + ba_ref[...]
    hp = jnp.dot(xp, wp_ref[...], preferred_element_type=jnp.float32) + bp_ref[...]
    ha_ref[...] = ha
    hp_ref[...] = hp
    ala_ref[...] = jnp.dot(ha, aa_ref[...], preferred_element_type=jnp.float32)
    alp_ref[...] = jnp.dot(hp, ap_ref[...], preferred_element_type=jnp.float32)


def _tc_proj(norm, xa, da, xp, dp, wa, ba, wp, bp, aa, ap):
    n, c = xa.shape
    f = wa.shape[1]
    bn = 2000
    grid = (n // bn,)
    xmap = lambda i: (i, 0)
    wmap = lambda i: (0, 0)
    in_specs = [
        pl.BlockSpec((bn, c), xmap),
        pl.BlockSpec((bn, 1), xmap),
        pl.BlockSpec((bn, c), xmap),
        pl.BlockSpec((bn, 1), xmap),
        pl.BlockSpec((c, f), wmap),
        pl.BlockSpec((1, f), wmap),
        pl.BlockSpec((c, f), wmap),
        pl.BlockSpec((1, f), wmap),
        pl.BlockSpec((f, 2), wmap),
        pl.BlockSpec((f, 2), wmap),
    ]
    out_specs = [
        pl.BlockSpec((bn, f), xmap),
        pl.BlockSpec((bn, f), xmap),
        pl.BlockSpec((bn, 2), xmap),
        pl.BlockSpec((bn, 2), xmap),
    ]
    out_shape = [
        jax.ShapeDtypeStruct((n, f), jnp.float32),
        jax.ShapeDtypeStruct((n, f), jnp.float32),
        jax.ShapeDtypeStruct((n, 2), jnp.float32),
        jax.ShapeDtypeStruct((n, 2), jnp.float32),
    ]
    return pl.pallas_call(
        functools.partial(_proj_body, norm),
        grid=grid, in_specs=in_specs, out_specs=out_specs, out_shape=out_shape,
    )(xa, da, xp, dp, wa, ba, wp, bp, aa, ap)


def _final_body(xa_ref, da_ref, xp_ref, dp_ref, w_ref, b_ref, mw_ref, mr_ref,
                oa_ref, op_ref, up_ref, ua_ref):
    xa = jnp.maximum(xa_ref[...] / (da_ref[...] + _EPS), 0.0)
    xp = jnp.maximum(xp_ref[...] / (dp_ref[...] + _EPS), 0.0)
    oa = jnp.dot(xa, w_ref[...], preferred_element_type=jnp.float32) + b_ref[...]
    op = jnp.dot(xp, w_ref[...], preferred_element_type=jnp.float32) + b_ref[...]
    oa_ref[...] = oa
    op_ref[...] = op
    up_ref[...] = jnp.dot(op, mw_ref[...], preferred_element_type=jnp.float32)
    ua_ref[...] = jnp.dot(oa, mr_ref[...], preferred_element_type=jnp.float32)


def _tc_final(xa, da, xp, dp, w, b, mw, mr):
    n, c = xa.shape
    bn = 2000
    grid = (n // bn,)
    xmap = lambda i: (i, 0)
    wmap = lambda i: (0, 0)
    in_specs = [
        pl.BlockSpec((bn, c), xmap),
        pl.BlockSpec((bn, 1), xmap),
        pl.BlockSpec((bn, c), xmap),
        pl.BlockSpec((bn, 1), xmap),
        pl.BlockSpec((c, 2), wmap),
        pl.BlockSpec((1, 2), wmap),
        pl.BlockSpec((2, 2), wmap),
        pl.BlockSpec((2, 2), wmap),
    ]
    out_specs = [pl.BlockSpec((bn, 2), xmap)] * 4
    out_shape = [jax.ShapeDtypeStruct((n, 2), jnp.float32)] * 4
    return pl.pallas_call(
        _final_body,
        grid=grid, in_specs=in_specs, out_specs=out_specs, out_shape=out_shape,
    )(xa, da, xp, dp, w, b, mw, mr)


@functools.lru_cache(maxsize=None)
def _mp_kernel(F, EP, NP):
    ET = EP // _NSUB
    nchunk = ET // _CH
    rows_per_tile = NP // _NSUB
    nout = rows_per_tile // _CH
    mesh = plsc.VectorSubcoreMesh(core_axis_name="c", subcore_axis_name="s",
                                  num_cores=_NCORE, num_subcores=_NSUB)

    def body(ha, hp, roww, colw, rowr, colr, asw, adw, asr, adr,
             aggp, denp, agga, dena,
             asrc_v, adst_v, row3_v, col3_v,
             w0_v, w1_v, w2_v, rows0_v, rows1_v, rows2_v,
             out_sp, den_sp,
             gsem0, gsem1, gsem2, ssem0, ssem1, ssem2,
             dsem0, dsem1, dsem2):
        s = lax.axis_index("s")
        c = lax.axis_index("c")
        zero16 = jnp.zeros((16,), jnp.float32)
        rv = (rows0_v, rows1_v, rows2_v)
        wv = (w0_v, w1_v, w2_v)
        gsems = (gsem0, gsem1, gsem2)
        ssems = (ssem0, ssem1, ssem2)
        dsems = (dsem0, dsem1, dsem2)

        def run(h_hbm, row_hbm, col_hbm, as_hbm, ad_hbm, agg_hbm, den_hbm):
            pltpu.sync_copy(as_hbm, asrc_v)
            pltpu.sync_copy(ad_hbm, adst_v)
            pltpu.sync_copy(row_hbm.at[pl.ds(s * nchunk, nchunk)], row3_v)
            pltpu.sync_copy(col_hbm.at[pl.ds(s * nchunk, nchunk)], col3_v)

            def zrow(e, _):
                for f in range(F // 16):
                    rows0_v[e, f * 16:(f + 1) * 16] = zero16
                return 0
            lax.fori_loop(0, _CH, zrow, 0)
            for g in range(_CH // 16):
                w0_v[g * 16:(g + 1) * 16] = zero16
            for k in range(nout):
                off = s * rows_per_tile + k * _CH
                pltpu.sync_copy(rows0_v, out_sp.at[pl.ds(off, _CH)])
                pltpu.sync_copy(w0_v, den_sp.at[pl.ds(off, _CH)])
            plsc.subcore_barrier()

            pltpu.async_copy(h_hbm.at[row3_v.at[0, 0]], rows0_v, gsem0)
            pltpu.async_copy(h_hbm.at[row3_v.at[1, 0]], rows1_v, gsem1)

            def den_drain(g, j):
                pltpu.make_async_copy(
                    wv[j], den_sp.at[col3_v.at[g, 0]], dsems[j]).wait()

            def gat_drain(g, j):
                pltpu.make_async_copy(h_hbm.at[row3_v.at[g, 0]], rv[j],
                                      gsems[j]).wait()

            def scat_drain(g, j):
                pltpu.make_async_copy(rv[j], out_sp.at[col3_v.at[g, 0]],
                                      ssems[j]).wait()

            def tri(i, _):
                for j in range(3):
                    g = 3 * i + j
                    w_v = wv[j]
                    rows_v = rv[j]

                    @pl.when(g >= 3)
                    def _():
                        den_drain(g - 3, j)
                    for gg in range(_CH // 16):
                        r16 = row3_v[g, 0, gg * 16:(gg + 1) * 16]
                        c16 = col3_v[g, 0, gg * 16:(gg + 1) * 16]
                        a = (plsc.load_gather(asrc_v, [r16])
                             + plsc.load_gather(adst_v, [c16]))
                        a = jnp.where(a > 0.0, a, 0.2 * a)
                        w_v[gg * 16:(gg + 1) * 16] = jnp.exp(a)
                    pltpu.async_copy(w_v, den_sp.at[col3_v.at[g, 0]],
                                     dsems[j], add=True)
                    gat_drain(g, j)

                    def scale(g2, _):
                        w16 = w_v[pl.ds(g2 * 16, 16)]
                        for e in range(16):
                            sw = w16[e]
                            idx = g2 * 16 + e
                            for f in range(F // 16):
                                rows_v[idx, f * 16:(f + 1) * 16] = (
                                    rows_v[idx, f * 16:(f + 1) * 16] * sw)
                        return 0
                    lax.fori_loop(0, _CH // 16, scale, 0)
                    pltpu.async_copy(rows_v, out_sp.at[col3_v.at[g, 0]],
                                     ssems[j], add=True)

                    jp = (j + 2) % 3

                    @pl.when(g >= 1)
                    def _():
                        scat_drain(g - 1, jp)

                    @pl.when(g + 2 < nchunk)
                    def _():
                        pltpu.async_copy(h_hbm.at[row3_v.at[g + 2, 0]],
                                         rv[jp], gsems[jp])
                return 0
            lax.fori_loop(0, nchunk // 3, tri, 0)
            scat_drain(nchunk - 1, (nchunk - 1) % 3)
            for dg in range(3):
                den_drain(nchunk - 3 + dg, (nchunk - 3 + dg) % 3)
            plsc.subcore_barrier()

            for k in range(nout):
                off = s * rows_per_tile + k * _CH
                pltpu.sync_copy(out_sp.at[pl.ds(off, _CH)], rows0_v)
                pltpu.sync_copy(rows0_v, agg_hbm.at[pl.ds(off, _CH)])
                pltpu.sync_copy(den_sp.at[pl.ds(off, _CH)], w0_v)
                pltpu.sync_copy(w0_v, den_hbm.at[pl.ds(off, _CH)])

        @pl.when(c == 0)
        def _():
            run(ha, roww, colw, asw, adw, aggp, denp)

        @pl.when(c == 1)
        def _():
            run(hp, rowr, colr, asr, adr, agga, dena)

    nch = EP // _NSUB // _CH
    return pl.kernel(
        body,
        out_type=[
            jax.ShapeDtypeStruct((NP, F), jnp.float32),
            jax.ShapeDtypeStruct((NP,), jnp.float32),
            jax.ShapeDtypeStruct((NP, F), jnp.float32),
            jax.ShapeDtypeStruct((NP,), jnp.float32),
        ],
        mesh=mesh,
        compiler_params=pltpu.CompilerParams(needs_layout_passes=False, use_tc_tiling_on_sc=False),
        scratch_types=[
            pltpu.VMEM((NP,), jnp.float32),
            pltpu.VMEM((NP,), jnp.float32),
            pltpu.VMEM((nch, 1, _CH), jnp.int32),
            pltpu.VMEM((nch, 1, _CH), jnp.int32),
            pltpu.VMEM((_CH,), jnp.float32),
            pltpu.VMEM((_CH,), jnp.float32),
            pltpu.VMEM((_CH,), jnp.float32),
            pltpu.VMEM((_CH, F), jnp.float32),
            pltpu.VMEM((_CH, F), jnp.float32),
            pltpu.VMEM((_CH, F), jnp.float32),
            pltpu.VMEM_SHARED((NP, F), jnp.float32),
            pltpu.VMEM_SHARED((NP,), jnp.float32),
        ] + [pltpu.SemaphoreType.DMA] * 9,
    )


@functools.lru_cache(maxsize=None)
def _score_kernel(N, LP):
    per = LP // _NSUB
    mesh = plsc.VectorSubcoreMesh(core_axis_name="c", subcore_axis_name="s",
                                  num_cores=_NCORE, num_subcores=_NSUB)

    def body(oa, up, op, ua, eliw, elir, out,
             head_v, tail_v, src_v, dst_v, res_v):
        s = lax.axis_index("s")
        c = lax.axis_index("c")

        def run(h_hbm, t_hbm, eli_hbm, out_row):
            pltpu.sync_copy(h_hbm, head_v)
            pltpu.sync_copy(t_hbm, tail_v)
            base = s * per
            pltpu.sync_copy(eli_hbm.at[0, pl.ds(base, per)], src_v)
            pltpu.sync_copy(eli_hbm.at[1, pl.ds(base, per)], dst_v)
            def grp(g, _):
                s16 = src_v[pl.ds(g * 16, 16)] * 2
                d16 = dst_v[pl.ds(g * 16, 16)] * 2
                h0 = plsc.load_gather(head_v, [s16])
                h1 = plsc.load_gather(head_v, [s16 + 1])
                u0 = plsc.load_gather(tail_v, [d16])
                u1 = plsc.load_gather(tail_v, [d16 + 1])
                res_v[pl.ds(g * 16, 16)] = h0 * u0 + h1 * u1
                return 0
            lax.fori_loop(0, per // 16, grp, 0)
            pltpu.sync_copy(res_v, out.at[out_row, pl.ds(base, per)])

        @pl.when(c == 0)
        def _():
            run(oa, up, eliw, 0)

        @pl.when(c == 1)
        def _():
            run(op, ua, elir, 1)

    return pl.kernel(
        body,
        out_type=jax.ShapeDtypeStruct((2, LP), jnp.float32),
        mesh=mesh,
        compiler_params=pltpu.CompilerParams(needs_layout_passes=False, use_tc_tiling_on_sc=False),
        scratch_types=[
            pltpu.VMEM((2 * N,), jnp.float32),
            pltpu.VMEM((2 * N,), jnp.float32),
            pltpu.VMEM((per,), jnp.int32),
            pltpu.VMEM((per,), jnp.int32),
            pltpu.VMEM((per,), jnp.float32),
        ],
    )


def _round_up(x, m):
    return (x + m - 1) // m * m


def _pad_rows(x, np_):
    return jnp.concatenate(
        [x, jnp.zeros((np_ - x.shape[0], x.shape[1]), x.dtype)])


def _pad_alpha(v, np_):
    return jnp.concatenate(
        [v, jnp.full((np_ - v.shape[0],), -1e30, jnp.float32)])


def _pad_edges(ei, ep, n, np_):
    e = ei.shape[1]
    fill = n + (jnp.arange(ep - e, dtype=jnp.int32) % (np_ - n))
    row = jnp.concatenate([ei[0], fill]).reshape(ep // _CH, 1, _CH)
    col = jnp.concatenate([ei[1], fill]).reshape(ep // _CH, 1, _CH)
    return row, col


def _layer_args(lp):
    wa = lp["proj"]["author"]["W"].T
    wp = lp["proj"]["paper"]["W"].T
    ba = lp["proj"]["author"]["b"][None, :]
    bp = lp["proj"]["paper"]["b"][None, :]
    aa = jnp.stack([lp["att"]["writes"]["src"], lp["att"]["rev"]["dst"]], axis=1)
    ap = jnp.stack([lp["att"]["writes"]["dst"], lp["att"]["rev"]["src"]], axis=1)
    return wa, ba, wp, bp, aa, ap


def kernel(x_author, x_paper, params, edge_index_writes, edge_index_rev,
           edge_label_index_writes, edge_label_index_rev):
    n = x_author.shape[0]
    e = edge_index_writes.shape[1]
    l = edge_label_index_writes.shape[1]
    np_ = _round_up(n + 1, _NSUB * _CH)
    ep = _round_up(e, 3 * _NSUB * _CH)
    lp_ = _round_up(l, _NSUB * 16)

    ones = jnp.ones((n, 1), jnp.float32)

    def mp_layer(h_a, h_p, al_a, al_p, F):
        ha_p = _pad_rows(h_a, np_)
        hp_p = _pad_rows(h_p, np_)
        asw = _pad_alpha(al_a[:, 0], np_)
        adw = _pad_alpha(al_p[:, 0], np_)
        asr = _pad_alpha(al_p[:, 1], np_)
        adr = _pad_alpha(al_a[:, 1], np_)
        roww, colw = _pad_edges(edge_index_writes, ep, n, np_)
        rowr, colr = _pad_edges(edge_index_rev, ep, n, np_)
        aggp, denp, agga, dena = _mp_kernel(F, ep, np_)(
            ha_p, hp_p, roww, colw, rowr, colr, asw, adw, asr, adr)
        return (agga[:n], dena[:n, None], aggp[:n], denp[:n, None])

    h_a, h_p, al_a, al_p = _tc_proj(False, x_author, ones, x_paper, ones,
                                    *_layer_args(params["l1"]))
    agga, dena, aggp, denp = mp_layer(h_a, h_p, al_a, al_p, h_a.shape[1])

    h_a, h_p, al_a, al_p = _tc_proj(True, agga, dena, aggp, denp,
                                    *_layer_args(params["l2"]))
    agga, dena, aggp, denp = mp_layer(h_a, h_p, al_a, al_p, h_a.shape[1])

    r = params["rel_emb"]
    mw = jnp.stack([jnp.stack([r[0, 0], -r[0, 1]]),
                    jnp.stack([r[0, 1], r[0, 0]])])
    mr = jnp.stack([jnp.stack([r[1, 0], -r[1, 1]]),
                    jnp.stack([r[1, 1], r[1, 0]])])
    oa, op, up, ua = _tc_final(agga, dena, aggp, denp,
                               params["post_W"].T, params["post_b"][None, :],
                               mw, mr)

    pad = jnp.zeros((2, lp_ - l), jnp.int32)
    eliw = jnp.concatenate([edge_label_index_writes, pad], axis=1)
    elir = jnp.concatenate([edge_label_index_rev, pad], axis=1)
    scores = _score_kernel(n, lp_)(oa.reshape(-1), up.reshape(-1),
                                   op.reshape(-1), ua.reshape(-1), eliw, elir)
    return scores[:, :l]


# speedup vs baseline: 60.2606x; 1.0337x over previous
import functools

import jax
import jax.numpy as jnp
from jax import lax
from jax.experimental import pallas as pl
from jax.experimental.pallas import tpu as pltpu
from jax.experimental.pallas import tpu_sc as plsc

_EPS = 1e-16
_CH = 128
_NSUB = 16
_NCORE = 2


def _proj_body(norm, xa_ref, da_ref, xp_ref, dp_ref, wa_ref, ba_ref, wp_ref,
               bp_ref, aa_ref, ap_ref, ha_ref, hp_ref, ala_ref, alp_ref):
    xa = xa_ref[...]
    xp = xp_ref[...]
    if norm:
        xa = jnp.maximum(xa / (da_ref[...] + _EPS), 0.0)
        xp = jnp.maximum(xp / (dp_ref[...] + _EPS), 0.0)
    ha = jnp.dot(xa, wa_ref[...], preferred_element_type=jnp.float32) + ba_ref[...]
    hp = jnp.dot(xp, wp_ref[...], preferred_element_type=jnp.float32) + bp_ref[...]
    ha_ref[...] = ha
    hp_ref[...] = hp
    ala_ref[...] = jnp.dot(ha, aa_ref[...], preferred_element_type=jnp.float32)
    alp_ref[...] = jnp.dot(hp, ap_ref[...], preferred_element_type=jnp.float32)


def _tc_proj(norm, xa, da, xp, dp, wa, ba, wp, bp, aa, ap):
    n, c = xa.shape
    f = wa.shape[1]
    bn = 2048 if n % 2048 == 0 else 2000
    grid = (n // bn,)
    xmap = lambda i: (i, 0)
    wmap = lambda i: (0, 0)
    in_specs = [
        pl.BlockSpec((bn, c), xmap),
        pl.BlockSpec((bn, 1), xmap),
        pl.BlockSpec((bn, c), xmap),
        pl.BlockSpec((bn, 1), xmap),
        pl.BlockSpec((c, f), wmap),
        pl.BlockSpec((1, f), wmap),
        pl.BlockSpec((c, f), wmap),
        pl.BlockSpec((1, f), wmap),
        pl.BlockSpec((f, 2), wmap),
        pl.BlockSpec((f, 2), wmap),
    ]
    out_specs = [
        pl.BlockSpec((bn, f), xmap),
        pl.BlockSpec((bn, f), xmap),
        pl.BlockSpec((bn, 2), xmap),
        pl.BlockSpec((bn, 2), xmap),
    ]
    out_shape = [
        jax.ShapeDtypeStruct((n, f), jnp.float32),
        jax.ShapeDtypeStruct((n, f), jnp.float32),
        jax.ShapeDtypeStruct((n, 2), jnp.float32),
        jax.ShapeDtypeStruct((n, 2), jnp.float32),
    ]
    return pl.pallas_call(
        functools.partial(_proj_body, norm),
        grid=grid, in_specs=in_specs, out_specs=out_specs, out_shape=out_shape,
    )(xa, da, xp, dp, wa, ba, wp, bp, aa, ap)


def _final_body(xa_ref, da_ref, xp_ref, dp_ref, w_ref, b_ref, mw_ref, mr_ref,
                oa_ref, op_ref, up_ref, ua_ref):
    xa = jnp.maximum(xa_ref[...] / (da_ref[...] + _EPS), 0.0)
    xp = jnp.maximum(xp_ref[...] / (dp_ref[...] + _EPS), 0.0)
    oa = jnp.dot(xa, w_ref[...], preferred_element_type=jnp.float32) + b_ref[...]
    op = jnp.dot(xp, w_ref[...], preferred_element_type=jnp.float32) + b_ref[...]
    oa_ref[...] = oa
    op_ref[...] = op
    up_ref[...] = jnp.dot(op, mw_ref[...], preferred_element_type=jnp.float32)
    ua_ref[...] = jnp.dot(oa, mr_ref[...], preferred_element_type=jnp.float32)


def _tc_final(xa, da, xp, dp, w, b, mw, mr):
    n, c = xa.shape
    bn = 2048 if n % 2048 == 0 else 2000
    grid = (n // bn,)
    xmap = lambda i: (i, 0)
    wmap = lambda i: (0, 0)
    in_specs = [
        pl.BlockSpec((bn, c), xmap),
        pl.BlockSpec((bn, 1), xmap),
        pl.BlockSpec((bn, c), xmap),
        pl.BlockSpec((bn, 1), xmap),
        pl.BlockSpec((c, 2), wmap),
        pl.BlockSpec((1, 2), wmap),
        pl.BlockSpec((2, 2), wmap),
        pl.BlockSpec((2, 2), wmap),
    ]
    out_specs = [pl.BlockSpec((bn, 2), xmap)] * 4
    out_shape = [jax.ShapeDtypeStruct((n, 2), jnp.float32)] * 4
    return pl.pallas_call(
        _final_body,
        grid=grid, in_specs=in_specs, out_specs=out_specs, out_shape=out_shape,
    )(xa, da, xp, dp, w, b, mw, mr)


@functools.lru_cache(maxsize=None)
def _mp_kernel(F, EP, NP):
    ET = EP // _NSUB
    nchunk = ET // _CH
    rows_per_tile = NP // _NSUB
    nout = rows_per_tile // _CH
    mesh = plsc.VectorSubcoreMesh(core_axis_name="c", subcore_axis_name="s",
                                  num_cores=_NCORE, num_subcores=_NSUB)

    def body(ha, hp, roww, colw, rowr, colr, asw, adw, asr, adr,
             aggp, denp, agga, dena,
             asrc_v, adst_v, row3_v, col3_v,
             w0_v, w1_v, w2_v, rows0_v, rows1_v, rows2_v,
             out_sp, den_sp,
             gsem0, gsem1, gsem2, ssem0, ssem1, ssem2,
             dsem0, dsem1, dsem2):
        s = lax.axis_index("s")
        c = lax.axis_index("c")
        zero16 = jnp.zeros((16,), jnp.float32)
        rv = (rows0_v, rows1_v, rows2_v)
        wv = (w0_v, w1_v, w2_v)
        gsems = (gsem0, gsem1, gsem2)
        ssems = (ssem0, ssem1, ssem2)
        dsems = (dsem0, dsem1, dsem2)

        def run(h_hbm, row_hbm, col_hbm, as_hbm, ad_hbm, agg_hbm, den_hbm):
            pltpu.sync_copy(as_hbm, asrc_v)
            pltpu.sync_copy(ad_hbm, adst_v)
            pltpu.sync_copy(row_hbm.at[pl.ds(s * nchunk, nchunk)], row3_v)
            pltpu.sync_copy(col_hbm.at[pl.ds(s * nchunk, nchunk)], col3_v)

            def zrow(e, _):
                for f in range(F // 16):
                    rows0_v[e, f * 16:(f + 1) * 16] = zero16
                return 0
            lax.fori_loop(0, _CH, zrow, 0)
            for g in range(_CH // 16):
                w0_v[g * 16:(g + 1) * 16] = zero16
            for k in range(nout):
                off = s * rows_per_tile + k * _CH
                pltpu.sync_copy(rows0_v, out_sp.at[pl.ds(off, _CH)])
                pltpu.sync_copy(w0_v, den_sp.at[pl.ds(off, _CH)])
            plsc.subcore_barrier()

            pltpu.async_copy(h_hbm.at[row3_v.at[0, 0]], rows0_v, gsem0)
            pltpu.async_copy(h_hbm.at[row3_v.at[1, 0]], rows1_v, gsem1)

            def den_drain(g, j):
                pltpu.make_async_copy(
                    wv[j], den_sp.at[col3_v.at[g, 0]], dsems[j]).wait()

            def gat_drain(g, j):
                pltpu.make_async_copy(h_hbm.at[row3_v.at[g, 0]], rv[j],
                                      gsems[j]).wait()

            def scat_drain(g, j):
                pltpu.make_async_copy(rv[j], out_sp.at[col3_v.at[g, 0]],
                                      ssems[j]).wait()

            def tri(i, _):
                for j in range(3):
                    g = 3 * i + j
                    w_v = wv[j]
                    rows_v = rv[j]

                    @pl.when(g >= 3)
                    def _():
                        den_drain(g - 3, j)
                    for gg in range(_CH // 16):
                        r16 = row3_v[g, 0, gg * 16:(gg + 1) * 16]
                        c16 = col3_v[g, 0, gg * 16:(gg + 1) * 16]
                        a = (plsc.load_gather(asrc_v, [r16])
                             + plsc.load_gather(adst_v, [c16]))
                        a = jnp.where(a > 0.0, a, 0.2 * a)
                        w_v[gg * 16:(gg + 1) * 16] = jnp.exp(a)
                    pltpu.async_copy(w_v, den_sp.at[col3_v.at[g, 0]],
                                     dsems[j], add=True)
                    gat_drain(g, j)

                    def scale(g2, _):
                        w16 = w_v[pl.ds(g2 * 16, 16)]
                        for e in range(16):
                            sw = w16[e]
                            idx = g2 * 16 + e
                            for f in range(F // 16):
                                rows_v[idx, f * 16:(f + 1) * 16] = (
                                    rows_v[idx, f * 16:(f + 1) * 16] * sw)
                        return 0
                    lax.fori_loop(0, _CH // 16, scale, 0)
                    pltpu.async_copy(rows_v, out_sp.at[col3_v.at[g, 0]],
                                     ssems[j], add=True)

                    jp = (j + 2) % 3

                    @pl.when(g >= 1)
                    def _():
                        scat_drain(g - 1, jp)

                    @pl.when(g + 2 < nchunk)
                    def _():
                        pltpu.async_copy(h_hbm.at[row3_v.at[g + 2, 0]],
                                         rv[jp], gsems[jp])
                return 0
            lax.fori_loop(0, nchunk // 3, tri, 0)
            scat_drain(nchunk - 1, (nchunk - 1) % 3)
            for dg in range(3):
                den_drain(nchunk - 3 + dg, (nchunk - 3 + dg) % 3)
            plsc.subcore_barrier()

            for k in range(nout):
                off = s * rows_per_tile + k * _CH
                pltpu.sync_copy(out_sp.at[pl.ds(off, _CH)], rows0_v)
                pltpu.sync_copy(rows0_v, agg_hbm.at[pl.ds(off, _CH)])
                pltpu.sync_copy(den_sp.at[pl.ds(off, _CH)], w0_v)
                pltpu.sync_copy(w0_v, den_hbm.at[pl.ds(off, _CH)])

        @pl.when(c == 0)
        def _():
            run(ha, roww, colw, asw, adw, aggp, denp)

        @pl.when(c == 1)
        def _():
            run(hp, rowr, colr, asr, adr, agga, dena)

    nch = EP // _NSUB // _CH
    return pl.kernel(
        body,
        out_type=[
            jax.ShapeDtypeStruct((NP, F), jnp.float32),
            jax.ShapeDtypeStruct((NP,), jnp.float32),
            jax.ShapeDtypeStruct((NP, F), jnp.float32),
            jax.ShapeDtypeStruct((NP,), jnp.float32),
        ],
        mesh=mesh,
        compiler_params=pltpu.CompilerParams(needs_layout_passes=False, use_tc_tiling_on_sc=False),
        scratch_types=[
            pltpu.VMEM((NP,), jnp.float32),
            pltpu.VMEM((NP,), jnp.float32),
            pltpu.VMEM((nch, 1, _CH), jnp.int32),
            pltpu.VMEM((nch, 1, _CH), jnp.int32),
            pltpu.VMEM((_CH,), jnp.float32),
            pltpu.VMEM((_CH,), jnp.float32),
            pltpu.VMEM((_CH,), jnp.float32),
            pltpu.VMEM((_CH, F), jnp.float32),
            pltpu.VMEM((_CH, F), jnp.float32),
            pltpu.VMEM((_CH, F), jnp.float32),
            pltpu.VMEM_SHARED((NP, F), jnp.float32),
            pltpu.VMEM_SHARED((NP,), jnp.float32),
        ] + [pltpu.SemaphoreType.DMA] * 9,
    )


@functools.lru_cache(maxsize=None)
def _score_kernel(N, LP):
    per = LP // _NSUB
    mesh = plsc.VectorSubcoreMesh(core_axis_name="c", subcore_axis_name="s",
                                  num_cores=_NCORE, num_subcores=_NSUB)

    def body(oa, up, op, ua, eliw, elir, out,
             head_v, tail_v, src_v, dst_v, res_v):
        s = lax.axis_index("s")
        c = lax.axis_index("c")

        def run(h_hbm, t_hbm, eli_hbm, out_row):
            pltpu.sync_copy(h_hbm, head_v)
            pltpu.sync_copy(t_hbm, tail_v)
            base = s * per
            pltpu.sync_copy(eli_hbm.at[0, pl.ds(base, per)], src_v)
            pltpu.sync_copy(eli_hbm.at[1, pl.ds(base, per)], dst_v)
            def grp(g, _):
                s16 = src_v[pl.ds(g * 16, 16)] * 2
                d16 = dst_v[pl.ds(g * 16, 16)] * 2
                h0 = plsc.load_gather(head_v, [s16])
                h1 = plsc.load_gather(head_v, [s16 + 1])
                u0 = plsc.load_gather(tail_v, [d16])
                u1 = plsc.load_gather(tail_v, [d16 + 1])
                res_v[pl.ds(g * 16, 16)] = h0 * u0 + h1 * u1
                return 0
            lax.fori_loop(0, per // 16, grp, 0)
            pltpu.sync_copy(res_v, out.at[out_row, pl.ds(base, per)])

        @pl.when(c == 0)
        def _():
            run(oa, up, eliw, 0)

        @pl.when(c == 1)
        def _():
            run(op, ua, elir, 1)

    return pl.kernel(
        body,
        out_type=jax.ShapeDtypeStruct((2, LP), jnp.float32),
        mesh=mesh,
        compiler_params=pltpu.CompilerParams(needs_layout_passes=False, use_tc_tiling_on_sc=False),
        scratch_types=[
            pltpu.VMEM((2 * N,), jnp.float32),
            pltpu.VMEM((2 * N,), jnp.float32),
            pltpu.VMEM((per,), jnp.int32),
            pltpu.VMEM((per,), jnp.int32),
            pltpu.VMEM((per,), jnp.float32),
        ],
    )


def _round_up(x, m):
    return (x + m - 1) // m * m


def _pad_rows(x, np_):
    return jnp.concatenate(
        [x, jnp.zeros((np_ - x.shape[0], x.shape[1]), x.dtype)])


def _pad_edges(ei, ep, n, np_):
    e = ei.shape[1]
    fill = n + (jnp.arange(ep - e, dtype=jnp.int32) % (np_ - n))
    row = jnp.concatenate([ei[0], fill]).reshape(ep // _CH, 1, _CH)
    col = jnp.concatenate([ei[1], fill]).reshape(ep // _CH, 1, _CH)
    return row, col


def _layer_args(lp):
    wa = lp["proj"]["author"]["W"].T
    wp = lp["proj"]["paper"]["W"].T
    ba = lp["proj"]["author"]["b"][None, :]
    bp = lp["proj"]["paper"]["b"][None, :]
    aa = jnp.stack([lp["att"]["writes"]["src"], lp["att"]["rev"]["dst"]], axis=1)
    ap = jnp.stack([lp["att"]["writes"]["dst"], lp["att"]["rev"]["src"]], axis=1)
    return wa, ba, wp, bp, aa, ap


def kernel(x_author, x_paper, params, edge_index_writes, edge_index_rev,
           edge_label_index_writes, edge_label_index_rev):
    n = x_author.shape[0]
    e = edge_index_writes.shape[1]
    l = edge_label_index_writes.shape[1]
    np_ = _round_up(n + 1, _NSUB * _CH)
    ep = _round_up(e, 3 * _NSUB * _CH)
    lp_ = _round_up(l, _NSUB * 16)

    xa = _pad_rows(x_author, np_)
    xp = _pad_rows(x_paper, np_)
    ones = jnp.ones((np_, 1), jnp.float32)
    roww, colw = _pad_edges(edge_index_writes, ep, n, np_)
    rowr, colr = _pad_edges(edge_index_rev, ep, n, np_)

    def mp_layer(h_a, h_p, al_a, al_p):
        return _mp_kernel(h_a.shape[1], ep, np_)(
            h_a, h_p, roww, colw, rowr, colr,
            al_a[:, 0], al_p[:, 0], al_p[:, 1], al_a[:, 1])

    h_a, h_p, al_a, al_p = _tc_proj(False, xa, ones, xp, ones,
                                    *_layer_args(params["l1"]))
    aggp, denp, agga, dena = mp_layer(h_a, h_p, al_a, al_p)

    h_a, h_p, al_a, al_p = _tc_proj(True, agga, dena[:, None], aggp,
                                    denp[:, None], *_layer_args(params["l2"]))
    aggp, denp, agga, dena = mp_layer(h_a, h_p, al_a, al_p)

    r = params["rel_emb"]
    mw = jnp.stack([jnp.stack([r[0, 0], -r[0, 1]]),
                    jnp.stack([r[0, 1], r[0, 0]])])
    mr = jnp.stack([jnp.stack([r[1, 0], -r[1, 1]]),
                    jnp.stack([r[1, 1], r[1, 0]])])
    oa, op, up, ua = _tc_final(agga, dena[:, None], aggp, denp[:, None],
                               params["post_W"].T, params["post_b"][None, :],
                               mw, mr)

    pad = jnp.zeros((2, lp_ - l), jnp.int32)
    eliw = jnp.concatenate([edge_label_index_writes, pad], axis=1)
    elir = jnp.concatenate([edge_label_index_rev, pad], axis=1)
    scores = _score_kernel(np_, lp_)(oa.reshape(-1), up.reshape(-1),
                                     op.reshape(-1), ua.reshape(-1),
                                     eliw, elir)
    return scores[:, :l]
